# Initial kernel scaffold; baseline (speedup 1.0000x reference)
import jax
import jax.numpy as jnp
from jax.experimental import pallas as pl


def kernel(edge_index, emb):
    raise NotImplementedError("write your pallas kernel here")


# speedup vs baseline: 6.7449x; 6.7449x over previous
import functools
import jax
import jax.numpy as jnp
from jax import lax
from jax.experimental import pallas as pl
from jax.experimental.pallas import tpu as pltpu
from jax.experimental.pallas import tpu_sc as plsc

N = 100000
D = 32
E = 1600000
L = 3
ALPHA = 1.0 / (L + 1)

NC = 2
NS = 16
NW = NC * NS

NP = 102400
NR = NP // 128
HALF = NP // NC
TRASH = 2048
ZROWS = HALF + TRASH

EPT = E // NW
EPS = E // NS
CH = 80
RPT = NR // NW

_f32 = jnp.float32
_i32 = jnp.int32
_mesh = plsc.VectorSubcoreMesh(core_axis_name="c", subcore_axis_name="s")


def _wid():
    return lax.axis_index("c") * NS + lax.axis_index("s")


def _iota16():
    return lax.iota(_i32, 16)


@functools.partial(
    pl.kernel,
    out_type=jax.ShapeDtypeStruct((NW, NR, 128), _f32),
    mesh=_mesh,
    compiler_params=pltpu.CompilerParams(needs_layout_passes=False, use_tc_tiling_on_sc=False),
    scratch_types=[
        pltpu.VMEM((NR, 128), _f32),
        pltpu.VMEM((CH,), _i32),
    ],
)
def _deg_kernel(col_hbm, partials, hist, colbuf):
    w = _wid()
    zf = jnp.zeros((16,), _f32)

    def zero_body(i, carry):
        for j in range(8):
            hist[i, pl.ds(j * 16, 16)] = zf
        return carry

    lax.fori_loop(0, NR, zero_body, None)

    iot = _iota16()

    def chunk(i, carry):
        base = w * EPT + i * CH
        pltpu.sync_copy(col_hbm.at[pl.ds(base, CH)], colbuf)
        for g in range(CH // 16):
            cols = colbuf[pl.ds(g * 16, 16)]
            sk, _sv = plsc.sort_key_val(cols, cols)
            prev = sk.at[jnp.maximum(iot - 1, 0)].get(
                mode="promise_in_bounds")
            nxt = sk.at[jnp.minimum(iot + 1, 15)].get(
                mode="promise_in_bounds")
            isstart = (iot == 0) | (sk != prev)
            isend = (iot == 15) | (sk != nxt)
            startpos = plsc.cummax(jnp.where(isstart, iot, 0))
            cnt = iot - startpos + 1
            r = lax.shift_right_logical(sk, 7)
            q = lax.bitwise_and(sk, 127)
            plsc.addupdate_scatter(hist, [r, q], cnt.astype(_f32), mask=isend)
        return carry

    lax.fori_loop(0, EPT // CH, chunk, None)
    pltpu.sync_copy(hist, partials.at[w])


@functools.partial(
    pl.kernel,
    out_type=(
        jax.ShapeDtypeStruct((NP, D), _f32),
        jax.ShapeDtypeStruct((NP,), _f32),
        jax.ShapeDtypeStruct((NP,), _f32),
    ),
    mesh=_mesh,
    compiler_params=pltpu.CompilerParams(needs_layout_passes=False, use_tc_tiling_on_sc=False),
    scratch_types=[
        pltpu.VMEM((NW, 128), _f32),
        pltpu.VMEM((128, D), _f32),
        pltpu.VMEM((128, D), _f32),
        pltpu.VMEM((128,), _f32),
        pltpu.VMEM((128,), _f32),
        pltpu.SemaphoreType.DMA,
    ],
)
def _scale_kernel(partials, emb, t_out, dis_out, inv_out,
                  pbuf, embbuf, tbuf, disbuf, invbuf, sem):
    w = _wid()

    def chunk(k, carry):
        rc = w * RPT + k
        nbase = rc * 128
        descs = [pltpu.async_copy(partials.at[p, rc], pbuf.at[p], sem)
                 for p in range(NW)]
        descs.append(pltpu.async_copy(emb.at[pl.ds(nbase, 128)], embbuf, sem))
        for dsc in descs:
            dsc.wait()
        for g in range(8):
            sl = pl.ds(g * 16, 16)
            deg = jnp.zeros((16,), _f32)
            for p in range(NW):
                deg = deg + pbuf[p, sl]
            bits = plsc.bitcast(deg, _i32)
            m = jnp.int32(0x5F3759DF) - lax.shift_right_logical(bits, 1)
            y = plsc.bitcast(m, _f32)
            for _ in range(3):
                y = y * (1.5 - 0.5 * deg * y * y)
            pos = deg > 0.0
            dis = jnp.where(pos, y, 0.0)
            disbuf[sl] = dis
            invbuf[sl] = dis * dis
            for j in range(16):
                b = dis.at[jnp.full((16,), j, _i32)].get(
                    mode="promise_in_bounds")
                n = g * 16 + j
                for h in range(2):
                    slh = pl.ds(h * 16, 16)
                    tbuf[n, slh] = embbuf[n, slh] * b
        pltpu.sync_copy(tbuf, t_out.at[pl.ds(nbase, 128)])
        pltpu.sync_copy(disbuf, dis_out.at[pl.ds(nbase, 128)])
        pltpu.sync_copy(invbuf, inv_out.at[pl.ds(nbase, 128)])
        return carry

    lax.fori_loop(0, RPT, chunk, None)


def _edge_scatter_phase(c, s, row_hbm, col_hbm, t_in, z,
                        rowbuf, colbuf, idxbuf, msgbuf, sem):
    iot = _iota16()
    cbase = c * HALF

    def chunk(i, carry):
        ebase = s * EPS + i * CH
        pltpu.sync_copy(row_hbm.at[pl.ds(ebase, CH)], rowbuf)
        pltpu.sync_copy(col_hbm.at[pl.ds(ebase, CH)], colbuf)
        gd = pltpu.async_copy(t_in.at[rowbuf], msgbuf, sem)
        for g in range(CH // 16):
            cols = colbuf[pl.ds(g * 16, 16)]
            loc = cols - cbase
            inr = (loc >= 0) & (loc < HALF)
            tr = HALF + lax.bitwise_and(s * 128 + i * CH + g * 16 + iot,
                                        TRASH - 1)
            idxbuf[pl.ds(g * 16, 16)] = jnp.where(inr, loc, tr)
        gd.wait()
        pltpu.sync_copy(msgbuf, z.at[idxbuf], add=True)
        return carry

    lax.fori_loop(0, EPS // CH, chunk, None)


def _zero_z(s, z, zerobuf):
    zf = jnp.zeros((16,), _f32)
    for n in range(64):
        zerobuf[n, pl.ds(0, 16)] = zf
        zerobuf[n, pl.ds(16, 16)] = zf
    zpt = ZROWS // NS

    def zb(i, carry):
        pltpu.sync_copy(zerobuf, z.at[pl.ds(s * zpt + i * 64, 64)])
        return carry

    lax.fori_loop(0, zpt // 64, zb, None)


@functools.partial(
    pl.kernel,
    out_type=(
        jax.ShapeDtypeStruct((NP, D), _f32),
        jax.ShapeDtypeStruct((NP, D), _f32),
    ),
    mesh=_mesh,
    compiler_params=pltpu.CompilerParams(needs_layout_passes=False, use_tc_tiling_on_sc=False),
    scratch_types=[
        pltpu.VMEM_SHARED((ZROWS, D), _f32),
        pltpu.VMEM((CH,), _i32),
        pltpu.VMEM((CH,), _i32),
        pltpu.VMEM((CH,), _i32),
        pltpu.VMEM((CH, D), _f32),
        pltpu.VMEM((64, D), _f32),
        pltpu.VMEM((64,), _f32),
        pltpu.VMEM((64,), _f32),
        pltpu.VMEM((64, D), _f32),
        pltpu.SemaphoreType.DMA,
    ],
)
def _layer_kernel(row_hbm, col_hbm, t_in, dis_hbm, inv_hbm,
                  t_next, y_out, z, rowbuf, colbuf, idxbuf, msgbuf,
                  zbuf, disbuf, invbuf, tbuf, sem):
    c = lax.axis_index("c")
    s = lax.axis_index("s")
    _zero_z(s, z, tbuf)
    plsc.subcore_barrier()
    _edge_scatter_phase(c, s, row_hbm, col_hbm, t_in, z,
                        rowbuf, colbuf, idxbuf, msgbuf, sem)
    plsc.subcore_barrier()
    cbase = c * HALF

    def ep(k, carry):
        lb = (s * RPT * 2 + k) * 64
        nbase = cbase + lb
        pltpu.sync_copy(z.at[pl.ds(lb, 64)], zbuf)
        pltpu.sync_copy(dis_hbm.at[pl.ds(nbase, 64)], disbuf)
        pltpu.sync_copy(inv_hbm.at[pl.ds(nbase, 64)], invbuf)
        for g in range(4):
            iv = invbuf[pl.ds(g * 16, 16)]
            for j in range(16):
                n = g * 16 + j
                bi = iv.at[jnp.full((16,), j, _i32)].get(
                    mode="promise_in_bounds")
                for h in range(2):
                    sl = pl.ds(h * 16, 16)
                    tbuf[n, sl] = zbuf[n, sl] * bi
        pltpu.sync_copy(tbuf, t_next.at[pl.ds(nbase, 64)])
        for g in range(4):
            dv = disbuf[pl.ds(g * 16, 16)]
            for j in range(16):
                n = g * 16 + j
                bd = dv.at[jnp.full((16,), j, _i32)].get(
                    mode="promise_in_bounds")
                for h in range(2):
                    sl = pl.ds(h * 16, 16)
                    tbuf[n, sl] = zbuf[n, sl] * bd
        pltpu.sync_copy(tbuf, y_out.at[pl.ds(nbase, 64)])
        return carry

    lax.fori_loop(0, RPT * 2, ep, None)


@functools.partial(
    pl.kernel,
    out_type=jax.ShapeDtypeStruct((NP, D), _f32),
    mesh=_mesh,
    compiler_params=pltpu.CompilerParams(needs_layout_passes=False, use_tc_tiling_on_sc=False),
    scratch_types=[
        pltpu.VMEM_SHARED((ZROWS, D), _f32),
        pltpu.VMEM((CH,), _i32),
        pltpu.VMEM((CH,), _i32),
        pltpu.VMEM((CH,), _i32),
        pltpu.VMEM((CH, D), _f32),
        pltpu.VMEM((64, D), _f32),
        pltpu.VMEM((64,), _f32),
        pltpu.VMEM((64, D), _f32),
        pltpu.VMEM((64, D), _f32),
        pltpu.VMEM((64, D), _f32),
        pltpu.SemaphoreType.DMA,
    ],
)
def _last_layer_kernel(row_hbm, col_hbm, t_in, dis_hbm, emb, y1, y2,
                       out_hbm, z, rowbuf, colbuf, idxbuf, msgbuf,
                       zbuf, disbuf, embbuf, y1buf, y2buf, sem):
    c = lax.axis_index("c")
    s = lax.axis_index("s")
    _zero_z(s, z, embbuf)
    plsc.subcore_barrier()
    _edge_scatter_phase(c, s, row_hbm, col_hbm, t_in, z,
                        rowbuf, colbuf, idxbuf, msgbuf, sem)
    plsc.subcore_barrier()
    cbase = c * HALF

    def ep(k, carry):
        lb = (s * RPT * 2 + k) * 64
        nbase = cbase + lb
        pltpu.sync_copy(z.at[pl.ds(lb, 64)], zbuf)
        pltpu.sync_copy(dis_hbm.at[pl.ds(nbase, 64)], disbuf)
        pltpu.sync_copy(emb.at[pl.ds(nbase, 64)], embbuf)
        pltpu.sync_copy(y1.at[pl.ds(nbase, 64)], y1buf)
        pltpu.sync_copy(y2.at[pl.ds(nbase, 64)], y2buf)
        for g in range(4):
            dv = disbuf[pl.ds(g * 16, 16)]
            for j in range(16):
                n = g * 16 + j
                bd = dv.at[jnp.full((16,), j, _i32)].get(
                    mode="promise_in_bounds")
                for h in range(2):
                    sl = pl.ds(h * 16, 16)
                    acc = embbuf[n, sl] + y1buf[n, sl] + y2buf[n, sl] \
                        + zbuf[n, sl] * bd
                    embbuf[n, sl] = acc * ALPHA
        pltpu.sync_copy(embbuf, out_hbm.at[pl.ds(nbase, 64)])
        return carry

    lax.fori_loop(0, RPT * 2, ep, None)


@functools.partial(
    pl.kernel,
    out_type=jax.ShapeDtypeStruct((E,), _f32),
    mesh=_mesh,
    compiler_params=pltpu.CompilerParams(needs_layout_passes=False, use_tc_tiling_on_sc=False),
    scratch_types=[
        pltpu.VMEM((CH,), _i32),
        pltpu.VMEM((CH,), _i32),
        pltpu.VMEM((CH, D), _f32),
        pltpu.VMEM((CH, D), _f32),
        pltpu.VMEM((CH,), _f32),
        pltpu.SemaphoreType.DMA,
    ],
)
def _score_kernel(row_hbm, col_hbm, out_tab, scores,
                  rowbuf, colbuf, abuf, bbuf, sbuf, sem):
    w = _wid()
    iot = _iota16()

    def chunk(i, carry):
        base = w * EPT + i * CH
        pltpu.sync_copy(row_hbm.at[pl.ds(base, CH)], rowbuf)
        pltpu.sync_copy(col_hbm.at[pl.ds(base, CH)], colbuf)
        da = pltpu.async_copy(out_tab.at[rowbuf], abuf, sem)
        db = pltpu.async_copy(out_tab.at[colbuf], bbuf, sem)
        da.wait()
        db.wait()
        for g in range(CH // 16):
            acc = jnp.zeros((16,), _f32)
            for j in range(16):
                e = g * 16 + j
                p = abuf[e, pl.ds(0, 16)] * bbuf[e, pl.ds(0, 16)] \
                    + abuf[e, pl.ds(16, 16)] * bbuf[e, pl.ds(16, 16)]
                acc = jnp.where(iot == j, jnp.sum(p), acc)
            sbuf[pl.ds(g * 16, 16)] = acc
        pltpu.sync_copy(sbuf, scores.at[pl.ds(base, CH)])
        return carry

    lax.fori_loop(0, EPT // CH, chunk, None)


def kernel(edge_index, emb):
    row = edge_index[0].astype(_i32)
    col = edge_index[1].astype(_i32)
    emb_pad = jnp.zeros((NP, D), _f32).at[:N].set(emb)
    partials = _deg_kernel(col)
    t0, dis, inv = _scale_kernel(partials, emb_pad)
    t1, y1 = _layer_kernel(row, col, t0, dis, inv)
    t2, y2 = _layer_kernel(row, col, t1, dis, inv)
    out = _last_layer_kernel(row, col, t2, dis, emb_pad, y1, y2)
    return _score_kernel(row, col, out)


# speedup vs baseline: 10.4439x; 1.5484x over previous
import functools
import jax
import jax.numpy as jnp
from jax import lax
from jax.experimental import pallas as pl
from jax.experimental.pallas import tpu as pltpu
from jax.experimental.pallas import tpu_sc as plsc

N = 100000
D = 32
E = 1600000
L = 3
ALPHA = 1.0 / (L + 1)

NC = 2
NS = 16
NW = NC * NS

NP = 102400
NR = NP // 128
HALF = NP // NC
TRASH = 2048
ZROWS = HALF + TRASH

EPT = E // NW
EPS = E // NS
CH = 80
RPT = NR // NW

_f32 = jnp.float32
_i32 = jnp.int32
_mesh = plsc.VectorSubcoreMesh(core_axis_name="c", subcore_axis_name="s")


def _wid():
    return lax.axis_index("c") * NS + lax.axis_index("s")


def _iota16():
    return lax.iota(_i32, 16)


@functools.partial(
    pl.kernel,
    out_type=jax.ShapeDtypeStruct((NW, NR, 128), _f32),
    mesh=_mesh,
    compiler_params=pltpu.CompilerParams(needs_layout_passes=False, use_tc_tiling_on_sc=False),
    scratch_types=[
        pltpu.VMEM((NR, 128), _f32),
        pltpu.VMEM((CH,), _i32),
        pltpu.VMEM((CH,), _i32),
        pltpu.SemaphoreType.DMA,
        pltpu.SemaphoreType.DMA,
    ],
)
def _deg_kernel(col_hbm, partials, hist, colA, colB, cA, cB):
    w = _wid()
    zf = jnp.zeros((16,), _f32)

    def zero_body(i, carry):
        for j in range(8):
            hist[i, pl.ds(j * 16, 16)] = zf
        return carry

    lax.fori_loop(0, NR, zero_body, None)

    iot = _iota16()

    def issue(j, colbuf, sem):
        base = jnp.minimum(w * EPT + j * CH, E - CH)
        pltpu.async_copy(col_hbm.at[pl.ds(base, CH)], colbuf, sem)

    def histo(colbuf):
        for g in range(CH // 16):
            cols = colbuf[pl.ds(g * 16, 16)]
            sk, _sv = plsc.sort_key_val(cols, cols)
            prev = sk.at[jnp.maximum(iot - 1, 0)].get(
                mode="promise_in_bounds")
            nxt = sk.at[jnp.minimum(iot + 1, 15)].get(
                mode="promise_in_bounds")
            isstart = (iot == 0) | (sk != prev)
            isend = (iot == 15) | (sk != nxt)
            startpos = plsc.cummax(jnp.where(isstart, iot, 0))
            cnt = iot - startpos + 1
            r = lax.shift_right_logical(sk, 7)
            q = lax.bitwise_and(sk, 127)
            plsc.addupdate_scatter(hist, [r, q], cnt.astype(_f32), mask=isend)

    issue(0, colA, cA)

    def pair(ip, carry):
        pltpu.make_async_copy(col_hbm.at[pl.ds(0, CH)], colA, cA).wait()
        issue(ip * 2 + 1, colB, cB)
        histo(colA)
        pltpu.make_async_copy(col_hbm.at[pl.ds(0, CH)], colB, cB).wait()
        issue(ip * 2 + 2, colA, cA)
        histo(colB)
        return carry

    npair = EPT // CH // 2
    lax.fori_loop(0, npair, pair, None)
    pltpu.make_async_copy(col_hbm.at[pl.ds(0, CH)], colA, cA).wait()
    histo(colA)
    pltpu.sync_copy(hist, partials.at[w])


@functools.partial(
    pl.kernel,
    out_type=(
        jax.ShapeDtypeStruct((NP, D), _f32),
        jax.ShapeDtypeStruct((NP,), _f32),
        jax.ShapeDtypeStruct((NP,), _f32),
    ),
    mesh=_mesh,
    compiler_params=pltpu.CompilerParams(needs_layout_passes=False, use_tc_tiling_on_sc=False),
    scratch_types=[
        pltpu.VMEM((NW, 128), _f32),
        pltpu.VMEM((128, D), _f32),
        pltpu.VMEM((128, D), _f32),
        pltpu.VMEM((128,), _f32),
        pltpu.VMEM((128,), _f32),
        pltpu.SemaphoreType.DMA,
    ],
)
def _scale_kernel(partials, emb, t_out, dis_out, inv_out,
                  pbuf, embbuf, tbuf, disbuf, invbuf, sem):
    w = _wid()

    def chunk(k, carry):
        rc = w * RPT + k
        nbase = rc * 128
        descs = [pltpu.async_copy(partials.at[p, rc], pbuf.at[p], sem)
                 for p in range(NW)]
        descs.append(pltpu.async_copy(emb.at[pl.ds(nbase, 128)], embbuf, sem))
        for dsc in descs:
            dsc.wait()
        for g in range(8):
            sl = pl.ds(g * 16, 16)
            deg = jnp.zeros((16,), _f32)
            for p in range(NW):
                deg = deg + pbuf[p, sl]
            bits = plsc.bitcast(deg, _i32)
            m = jnp.int32(0x5F3759DF) - lax.shift_right_logical(bits, 1)
            y = plsc.bitcast(m, _f32)
            for _ in range(3):
                y = y * (1.5 - 0.5 * deg * y * y)
            pos = deg > 0.0
            dis = jnp.where(pos, y, 0.0)
            disbuf[sl] = dis
            invbuf[sl] = dis * dis
            for j in range(16):
                b = dis.at[jnp.full((16,), j, _i32)].get(
                    mode="promise_in_bounds")
                n = g * 16 + j
                for h in range(2):
                    slh = pl.ds(h * 16, 16)
                    tbuf[n, slh] = embbuf[n, slh] * b
        pltpu.sync_copy(tbuf, t_out.at[pl.ds(nbase, 128)])
        pltpu.sync_copy(disbuf, dis_out.at[pl.ds(nbase, 128)])
        pltpu.sync_copy(invbuf, inv_out.at[pl.ds(nbase, 128)])
        return carry

    lax.fori_loop(0, RPT, chunk, None)


def _edge_scatter_phase(c, s, row2d, col2d, t_in, z,
                        rowbig, colbig, idxbig, msgA, msgB, gsem):
    iot = _iota16()
    cbase = c * HALF
    SUB = 5
    nsuper = EPS // CH // SUB

    def outer(o, carry):
        cbchunk = s * (EPS // CH) + o * SUB
        pltpu.sync_copy(row2d.at[pl.ds(cbchunk, SUB)], rowbig)
        pltpu.sync_copy(col2d.at[pl.ds(cbchunk, SUB)], colbig)
        for k in range(SUB):
            for g in range(CH // 16):
                cols = colbig[k, pl.ds(g * 16, 16)]
                loc = cols - cbase
                inr = (loc >= 0) & (loc < HALF)
                tr = HALF + lax.bitwise_and(
                    s * 128 + (o * SUB + k) * CH + g * 16 + iot, TRASH - 1)
                idxbig[k, pl.ds(g * 16, 16)] = jnp.where(inr, loc, tr)
        for k in range(SUB):
            gd = pltpu.async_copy(t_in.at[rowbig.at[k]], msgA, gsem)
            gd.wait()
            pltpu.sync_copy(msgA, z.at[idxbig.at[k]], add=True)
        return carry

    lax.fori_loop(0, nsuper, outer, None)


def _zero_z(s, z, zerobuf):
    zf = jnp.zeros((16,), _f32)
    for n in range(64):
        zerobuf[n, pl.ds(0, 16)] = zf
        zerobuf[n, pl.ds(16, 16)] = zf
    zpt = ZROWS // NS

    def zb(i, carry):
        pltpu.sync_copy(zerobuf, z.at[pl.ds(s * zpt + i * 64, 64)])
        return carry

    lax.fori_loop(0, zpt // 64, zb, None)


@functools.partial(
    pl.kernel,
    out_type=(
        jax.ShapeDtypeStruct((NP, D), _f32),
        jax.ShapeDtypeStruct((NP, D), _f32),
    ),
    mesh=_mesh,
    compiler_params=pltpu.CompilerParams(needs_layout_passes=False, use_tc_tiling_on_sc=False),
    scratch_types=[
        pltpu.VMEM_SHARED((ZROWS, D), _f32),
        pltpu.VMEM((5, CH), _i32),
        pltpu.VMEM((5, CH), _i32),
        pltpu.VMEM((5, CH), _i32),
        pltpu.VMEM((CH, D), _f32),
        pltpu.VMEM((CH, D), _f32),
        pltpu.VMEM((64, D), _f32),
        pltpu.VMEM((64,), _f32),
        pltpu.VMEM((64,), _f32),
        pltpu.VMEM((64, D), _f32),
        pltpu.SemaphoreType.DMA,
        pltpu.SemaphoreType.DMA,
    ],
)
def _layer_kernel(row2d, col2d, t_in, dis_hbm, inv_hbm,
                  t_next, y_out, z, rowbig, colbig, idxbig, msgA, msgB,
                  zbuf, disbuf, invbuf, tbuf, gA, gB):
    c = lax.axis_index("c")
    s = lax.axis_index("s")
    _zero_z(s, z, tbuf)
    plsc.subcore_barrier()
    _edge_scatter_phase(c, s, row2d, col2d, t_in, z,
                        rowbig, colbig, idxbig, msgA, msgB, gA)
    plsc.subcore_barrier()
    cbase = c * HALF

    def ep(k, carry):
        lb = (s * RPT * 2 + k) * 64
        nbase = cbase + lb
        pltpu.sync_copy(z.at[pl.ds(lb, 64)], zbuf)
        pltpu.sync_copy(dis_hbm.at[pl.ds(nbase, 64)], disbuf)
        pltpu.sync_copy(inv_hbm.at[pl.ds(nbase, 64)], invbuf)
        for g in range(4):
            iv = invbuf[pl.ds(g * 16, 16)]
            for j in range(16):
                n = g * 16 + j
                bi = iv.at[jnp.full((16,), j, _i32)].get(
                    mode="promise_in_bounds")
                for h in range(2):
                    sl = pl.ds(h * 16, 16)
                    tbuf[n, sl] = zbuf[n, sl] * bi
        pltpu.sync_copy(tbuf, t_next.at[pl.ds(nbase, 64)])
        for g in range(4):
            dv = disbuf[pl.ds(g * 16, 16)]
            for j in range(16):
                n = g * 16 + j
                bd = dv.at[jnp.full((16,), j, _i32)].get(
                    mode="promise_in_bounds")
                for h in range(2):
                    sl = pl.ds(h * 16, 16)
                    tbuf[n, sl] = zbuf[n, sl] * bd
        pltpu.sync_copy(tbuf, y_out.at[pl.ds(nbase, 64)])
        return carry

    lax.fori_loop(0, RPT * 2, ep, None)


@functools.partial(
    pl.kernel,
    out_type=jax.ShapeDtypeStruct((NP, D), _f32),
    mesh=_mesh,
    compiler_params=pltpu.CompilerParams(needs_layout_passes=False, use_tc_tiling_on_sc=False),
    scratch_types=[
        pltpu.VMEM_SHARED((ZROWS, D), _f32),
        pltpu.VMEM((5, CH), _i32),
        pltpu.VMEM((5, CH), _i32),
        pltpu.VMEM((5, CH), _i32),
        pltpu.VMEM((CH, D), _f32),
        pltpu.VMEM((CH, D), _f32),
        pltpu.VMEM((64, D), _f32),
        pltpu.VMEM((64,), _f32),
        pltpu.VMEM((64, D), _f32),
        pltpu.VMEM((64, D), _f32),
        pltpu.VMEM((64, D), _f32),
        pltpu.SemaphoreType.DMA,
        pltpu.SemaphoreType.DMA,
    ],
)
def _last_layer_kernel(row2d, col2d, t_in, dis_hbm, emb, y1, y2,
                       out_hbm, z, rowbig, colbig, idxbig, msgA, msgB,
                       zbuf, disbuf, embbuf, y1buf, y2buf, gA, gB):
    c = lax.axis_index("c")
    s = lax.axis_index("s")
    _zero_z(s, z, embbuf)
    plsc.subcore_barrier()
    _edge_scatter_phase(c, s, row2d, col2d, t_in, z,
                        rowbig, colbig, idxbig, msgA, msgB, gA)
    plsc.subcore_barrier()
    cbase = c * HALF

    def ep(k, carry):
        lb = (s * RPT * 2 + k) * 64
        nbase = cbase + lb
        pltpu.sync_copy(z.at[pl.ds(lb, 64)], zbuf)
        pltpu.sync_copy(dis_hbm.at[pl.ds(nbase, 64)], disbuf)
        pltpu.sync_copy(emb.at[pl.ds(nbase, 64)], embbuf)
        pltpu.sync_copy(y1.at[pl.ds(nbase, 64)], y1buf)
        pltpu.sync_copy(y2.at[pl.ds(nbase, 64)], y2buf)
        for g in range(4):
            dv = disbuf[pl.ds(g * 16, 16)]
            for j in range(16):
                n = g * 16 + j
                bd = dv.at[jnp.full((16,), j, _i32)].get(
                    mode="promise_in_bounds")
                for h in range(2):
                    sl = pl.ds(h * 16, 16)
                    acc = embbuf[n, sl] + y1buf[n, sl] + y2buf[n, sl] \
                        + zbuf[n, sl] * bd
                    embbuf[n, sl] = acc * ALPHA
        pltpu.sync_copy(embbuf, out_hbm.at[pl.ds(nbase, 64)])
        return carry

    lax.fori_loop(0, RPT * 2, ep, None)


@functools.partial(
    pl.kernel,
    out_type=jax.ShapeDtypeStruct((E,), _f32),
    mesh=_mesh,
    compiler_params=pltpu.CompilerParams(needs_layout_passes=False, use_tc_tiling_on_sc=False),
    scratch_types=[
        pltpu.VMEM((CH,), _i32),
        pltpu.VMEM((CH,), _i32),
        pltpu.VMEM((CH, D), _f32),
        pltpu.VMEM((CH, D), _f32),
        pltpu.VMEM((CH,), _f32),
        pltpu.VMEM((CH,), _i32),
        pltpu.VMEM((CH,), _i32),
        pltpu.VMEM((CH, D), _f32),
        pltpu.VMEM((CH, D), _f32),
        pltpu.VMEM((CH,), _f32),
        pltpu.SemaphoreType.DMA,
        pltpu.SemaphoreType.DMA,
        pltpu.SemaphoreType.DMA,
        pltpu.SemaphoreType.DMA,
        pltpu.SemaphoreType.DMA,
        pltpu.SemaphoreType.DMA,
    ],
)
def _score_kernel(row_hbm, col_hbm, out_tab, scores,
                  rowA, colA, abufA, bbufA, sbufA,
                  rowB, colB, abufB, bbufB, sbufB,
                  cA, cB, gA, gB, sA, sB):
    w = _wid()
    iot = _iota16()

    def issue_idx(j, rowbuf, colbuf, sem):
        base = jnp.minimum(w * EPT + j * CH, E - CH)
        pltpu.async_copy(row_hbm.at[pl.ds(base, CH)], rowbuf, sem)
        pltpu.async_copy(col_hbm.at[pl.ds(base, CH)], colbuf, sem)

    def wait_idx(rowbuf, colbuf, sem):
        pltpu.make_async_copy(row_hbm.at[pl.ds(0, CH)], rowbuf, sem).wait()
        pltpu.make_async_copy(col_hbm.at[pl.ds(0, CH)], colbuf, sem).wait()

    def dots(abuf, bbuf, sbuf):
        for g in range(CH // 16):
            acc = jnp.zeros((16,), _f32)
            for j in range(16):
                e = g * 16 + j
                p = abuf[e, pl.ds(0, 16)] * bbuf[e, pl.ds(0, 16)] \
                    + abuf[e, pl.ds(16, 16)] * bbuf[e, pl.ds(16, 16)]
                acc = jnp.where(iot == j, jnp.sum(p), acc)
            sbuf[pl.ds(g * 16, 16)] = acc

    def half(i, rowbuf, colbuf, abuf, bbuf, sbuf, csem, gsem, ssem,
             pf, pfbufs, pfsem):
        pltpu.make_async_copy(sbuf, scores.at[pl.ds(0, CH)], ssem).wait()
        wait_idx(rowbuf, colbuf, csem)
        da = pltpu.async_copy(out_tab.at[rowbuf], abuf, gsem)
        db = pltpu.async_copy(out_tab.at[colbuf], bbuf, gsem)
        issue_idx(pf, pfbufs[0], pfbufs[1], pfsem)
        da.wait()
        db.wait()
        dots(abuf, bbuf, sbuf)
        pltpu.async_copy(sbuf, scores.at[pl.ds(w * EPT + i * CH, CH)], ssem)

    pltpu.async_copy(scores.at[pl.ds(w * EPT, CH)], sbufA, sA)
    pltpu.async_copy(scores.at[pl.ds(w * EPT, CH)], sbufB, sB)
    issue_idx(0, rowA, colA, cA)

    def pair(ip, carry):
        iA = ip * 2
        half(iA, rowA, colA, abufA, bbufA, sbufA, cA, gA, sA,
             iA + 1, (rowB, colB), cB)
        half(iA + 1, rowB, colB, abufB, bbufB, sbufB, cB, gB, sB,
             iA + 2, (rowA, colA), cA)
        return carry

    npair = EPT // CH // 2
    lax.fori_loop(0, npair, pair, None)
    tail = npair * 2
    half(tail, rowA, colA, abufA, bbufA, sbufA, cA, gA, sA,
         tail, (rowB, colB), cB)
    pltpu.make_async_copy(sbufA, scores.at[pl.ds(0, CH)], sA).wait()
    pltpu.make_async_copy(sbufB, scores.at[pl.ds(0, CH)], sB).wait()
    wait_idx(rowB, colB, cB)


def kernel(edge_index, emb):
    row = edge_index[0].astype(_i32)
    col = edge_index[1].astype(_i32)
    row2d = row.reshape(E // CH, CH)
    col2d = col.reshape(E // CH, CH)
    emb_pad = jnp.zeros((NP, D), _f32).at[:N].set(emb)
    partials = _deg_kernel(col)
    t0, dis, inv = _scale_kernel(partials, emb_pad)
    t1, y1 = _layer_kernel(row2d, col2d, t0, dis, inv)
    t2, y2 = _layer_kernel(row2d, col2d, t1, dis, inv)
    out = _last_layer_kernel(row2d, col2d, t2, dis, emb_pad, y1, y2)
    return _score_kernel(row, col, out)
